# baseline (speedup 1.0000x reference)
import jax
import jax.numpy as jnp
from jax import lax
from jax.experimental import pallas as pl
from jax.experimental.pallas import tpu as pltpu
from jax.experimental.pallas import tpu_sc as plsc

_K = 64
_NEG = -1e30
_R, _C = 128, 32768
_NV = _C // 16
_GROUPS = 64
_VPG = _NV // _GROUPS
_ROWS_PER_TEC = 4


def _key(v):
    b = plsc.bitcast(v, jnp.int32)
    return b ^ ((b >> 31) & jnp.int32(0x7FFFFFFF))


def _unkey_splat(t):
    ts = jnp.broadcast_to(t, (16,))
    return plsc.bitcast(ts ^ ((ts >> 31) & jnp.int32(0x7FFFFFFF)), jnp.float32)


def _scalar(v16):
    return lax.reduce_max(v16, axes=(0,))


def _sc_body(scores_hbm, out_hbm, rowa, rowb, candv, gmbuf,
             in_sem_a, in_sem_b, out_sem_a, out_sem_b):
    wid = lax.axis_index("s") * 2 + lax.axis_index("c")
    neg = jnp.full((16,), _NEG, jnp.float32)
    lane16 = lax.iota(jnp.int32, 16)
    bufs = [rowa, rowb]
    in_sems = [in_sem_a, in_sem_b]
    out_sems = [out_sem_a, out_sem_b]

    base_row = wid * _ROWS_PER_TEC
    in_flight = {0: pltpu.async_copy(scores_hbm.at[base_row], rowa, in_sem_a)}
    out_flight = {}

    for r in range(_ROWS_PER_TEC):
        rowbuf = bufs[r % 2]
        in_flight.pop(r).wait()

        lane64 = lane16 * 64

        def grp_body(g, carry):
            gms = [rowbuf[pl.ds(g * 512 + u * 16, 16)] for u in range(8)]
            for u in range(8, _VPG):
                gms[u % 8] = jnp.maximum(
                    gms[u % 8], rowbuf[pl.ds(g * 512 + u * 16, 16)])
            gm = gms[0]
            for u in range(1, 8):
                gm = jnp.maximum(gm, gms[u])
            plsc.store_scatter(gmbuf, [lane64 + g], gm)
            return carry
        lax.fori_loop(0, _GROUPS, grp_body, 0)

        colmax = [gmbuf[pl.ds(c * 16, 16)] for c in range(4)]
        for l in range(1, 16):
            for c in range(4):
                colmax[c] = jnp.maximum(colmax[c],
                                        gmbuf[pl.ds(l * 64 + c * 16, 16)])
        t_low = lax.reduce_max(
            -jnp.maximum(jnp.maximum(-colmax[0], -colmax[1]),
                         jnp.maximum(-colmax[2], -colmax[3])), axes=(0,))
        t_low = -t_low
        gmax = lax.reduce_max(
            jnp.maximum(jnp.maximum(colmax[0], colmax[1]),
                        jnp.maximum(colmax[2], colmax[3])), axes=(0,))
        tl_vec = jnp.broadcast_to(t_low, (16,))

        if r + 1 < _ROWS_PER_TEC:
            nxt = (r + 1) % 2
            if r - 1 in out_flight:
                out_flight.pop(r - 1).wait()
            in_flight[r + 1] = pltpu.async_copy(
                scores_hbm.at[base_row + r + 1], bufs[nxt], in_sems[nxt])

        def collect(i, cnt):
            xs, msks, pcs = [], [], []
            for u in range(16):
                x = rowbuf[pl.ds((i * 16 + u) * 16, 16)]
                m = x >= tl_vec
                xs.append(x)
                msks.append(m)
                pcs.append(plsc.all_reduce_population_count(m)[0])
            offs = [cnt]
            for u in range(15):
                offs.append(offs[u] + pcs[u])
            for u in range(16):
                plsc.store_compressed(candv.at[pl.ds(offs[u], 16)],
                                      xs[u], mask=msks[u])
            return offs[15] + pcs[15]
        cnt = lax.fori_loop(0, _NV // 16, collect, jnp.int32(0))

        inf_pad = jnp.full((16,), -jnp.inf, jnp.float32)
        true16 = jnp.full((16,), True, jnp.bool_)
        for p in range(4):
            plsc.store_compressed(candv.at[pl.ds(cnt + p * 16, 16)],
                                  inf_pad, mask=true16)
        nv4 = (cnt + 63) // 64

        def count_ge2(t1, t2):
            def cb(j, acc):
                a1, a2 = acc
                for u in range(4):
                    kv = _key(candv[pl.ds((j * 4 + u) * 16, 16)])
                    a1 = a1 + (kv >= t1).astype(jnp.int32)
                    a2 = a2 + (kv >= t2).astype(jnp.int32)
                return a1, a2
            p1, p2 = lax.fori_loop(
                0, nv4, cb,
                (jnp.zeros((16,), jnp.int32), jnp.zeros((16,), jnp.int32)))
            return (lax.reduce_max(plsc.cumsum(p1), axes=(0,)),
                    lax.reduce_max(plsc.cumsum(p2), axes=(0,)))

        def _cavg(a, b):
            return (a & b) + ((a ^ b) >> 1)

        lo0 = _scalar(_key(tl_vec)) - 1
        hi0 = _scalar(_key(jnp.broadcast_to(gmax, (16,)))) + 1

        def bs_cond(state):
            return state[0] < state[1]

        def bs_body(state):
            lo, hi, c_lo = state
            mid = _cavg(lo + 1, hi)
            m1 = _cavg(lo + 1, mid - 1)
            m1 = jnp.maximum(m1, lo + 1)
            c1, c2 = count_ge2(m1, mid)
            ge2 = c2 >= _K
            ge1 = c1 >= _K
            lo = jnp.where(ge2, mid, jnp.where(ge1, m1, lo))
            hi = jnp.where(ge2, hi, jnp.where(ge1, mid - 1, m1 - 1))
            c_lo = jnp.where(ge2, c2, jnp.where(ge1, c1, c_lo))
            return lo, hi, c_lo
        t, _unused, total = lax.while_loop(bs_cond, bs_body, (lo0, hi0, cnt))

        def count_ge(tq):
            def cb(j, acc):
                for u in range(4):
                    kv = _key(candv[pl.ds((j * 4 + u) * 16, 16)])
                    acc = acc + (kv >= tq).astype(jnp.int32)
                return acc
            part = lax.fori_loop(0, nv4, cb, jnp.zeros((16,), jnp.int32))
            return lax.reduce_max(plsc.cumsum(part), axes=(0,))
        fast = (total == _K) & (t != 0) & (t != -1)

        @pl.when(fast)
        def _():
            tf = _unkey_splat(t)

            def emit_fast(i, carry):
                for u in range(16):
                    sl = pl.ds((i * 16 + u) * 16, 16)
                    x = rowbuf[sl]
                    rowbuf[sl] = jnp.where(x >= tf, x, neg)
                return carry
            lax.fori_loop(0, _NV // 16, emit_fast, 0)

        @pl.when(jnp.logical_not(fast))
        def _():
            m_eq = _K - count_ge(t + 1)

            def emit(i, eq_seen):
                for u in range(4):
                    sl = pl.ds((i * 4 + u) * 16, 16)
                    x = rowbuf[sl]
                    kv = _key(x)
                    eq = kv == t
                    inc = plsc.cumsum(eq.astype(jnp.int32))
                    keep = (kv > t) | (eq & ((eq_seen + inc) <= m_eq))
                    rowbuf[sl] = jnp.where(keep, x, neg)
                    eq_seen = eq_seen + plsc.all_reduce_population_count(eq)[0]
                return eq_seen
            lax.fori_loop(0, _NV // 4, emit, jnp.int32(0))

        out_flight[r] = pltpu.async_copy(
            rowbuf, out_hbm.at[base_row + r], out_sems[r % 2])

    out_flight.pop(_ROWS_PER_TEC - 2).wait()
    out_flight.pop(_ROWS_PER_TEC - 1).wait()


def kernel(scores, k):
    mesh = plsc.VectorSubcoreMesh(core_axis_name="c", subcore_axis_name="s",
                                  num_cores=2, num_subcores=16)
    out = pl.kernel(
        _sc_body,
        out_type=jax.ShapeDtypeStruct((_R, _C), jnp.float32),
        mesh=mesh,
        compiler_params=pltpu.CompilerParams(needs_layout_passes=False),
        scratch_types=[
            pltpu.VMEM((_C,), jnp.float32),
            pltpu.VMEM((_C,), jnp.float32),
            pltpu.VMEM((_C + 64,), jnp.float32),
            pltpu.VMEM((1024,), jnp.float32),
            pltpu.SemaphoreType.DMA,
            pltpu.SemaphoreType.DMA,
            pltpu.SemaphoreType.DMA,
            pltpu.SemaphoreType.DMA,
        ],
    )(scores)
    return out + (k * 0)


# speedup vs baseline: 7.4349x; 7.4349x over previous
import jax
import jax.numpy as jnp
from jax import lax
from jax.experimental import pallas as pl
from jax.experimental.pallas import tpu as pltpu
from jax.experimental.pallas import tpu_sc as plsc

_K = 64
_NEG = -1e30
_R, _C = 128, 32768
_NV = _C // 16
_GROUPS = 64
_VPG = _NV // _GROUPS
_ROWS_PER_TEC = 4


def _key(v):
    b = plsc.bitcast(v, jnp.int32)
    return b ^ ((b >> 31) & jnp.int32(0x7FFFFFFF))


def _unkey_splat(t):
    ts = jnp.broadcast_to(t, (16,))
    return plsc.bitcast(ts ^ ((ts >> 31) & jnp.int32(0x7FFFFFFF)), jnp.float32)


def _scalar(v16):
    return lax.reduce_max(v16, axes=(0,))


def _sc_body(scores_hbm, out_hbm, rowa, rowb, candv,
             in_sem_a, in_sem_b, out_sem_a, out_sem_b):
    wid = lax.axis_index("s") * 2 + lax.axis_index("c")
    neg = jnp.full((16,), _NEG, jnp.float32)
    bufs = [rowa, rowb]
    in_sems = [in_sem_a, in_sem_b]
    out_sems = [out_sem_a, out_sem_b]

    base_row = wid * _ROWS_PER_TEC
    in_flight = {0: pltpu.async_copy(scores_hbm.at[base_row], rowa, in_sem_a)}
    out_flight = {}

    for r in range(_ROWS_PER_TEC):
        rowbuf = bufs[r % 2]
        in_flight.pop(r).wait()

        def grp_body(g, carry):
            t_low, gmax = carry
            gms = [rowbuf[pl.ds(g * 512 + u * 16, 16)] for u in range(8)]
            for u in range(8, _VPG):
                gms[u % 8] = jnp.maximum(
                    gms[u % 8], rowbuf[pl.ds(g * 512 + u * 16, 16)])
            gm = gms[0]
            for u in range(1, 8):
                gm = jnp.maximum(gm, gms[u])
            gmx = lax.reduce_max(gm, axes=(0,))
            return jnp.minimum(t_low, gmx), jnp.maximum(gmax, gmx)
        t_low, gmax = lax.fori_loop(
            0, _GROUPS, grp_body,
            (jnp.float32(jnp.inf), jnp.float32(-jnp.inf)))
        tl_vec = jnp.broadcast_to(t_low, (16,))

        if r + 1 < _ROWS_PER_TEC:
            nxt = (r + 1) % 2
            if r - 1 in out_flight:
                out_flight.pop(r - 1).wait()
            in_flight[r + 1] = pltpu.async_copy(
                scores_hbm.at[base_row + r + 1], bufs[nxt], in_sems[nxt])

        def collect(i, cnt):
            xs, msks, pcs = [], [], []
            for u in range(16):
                x = rowbuf[pl.ds((i * 16 + u) * 16, 16)]
                m = x >= tl_vec
                xs.append(x)
                msks.append(m)
                pcs.append(plsc.all_reduce_population_count(m)[0])
            offs = [cnt]
            for u in range(15):
                offs.append(offs[u] + pcs[u])
            for u in range(16):
                plsc.store_compressed(candv.at[pl.ds(offs[u], 16)],
                                      xs[u], mask=msks[u])
            return offs[15] + pcs[15]
        cnt = lax.fori_loop(0, _NV // 16, collect, jnp.int32(0))

        inf_pad = jnp.full((16,), -jnp.inf, jnp.float32)
        true16 = jnp.full((16,), True, jnp.bool_)
        for p in range(4):
            plsc.store_compressed(candv.at[pl.ds(cnt + p * 16, 16)],
                                  inf_pad, mask=true16)
        nv4 = (cnt + 63) // 64

        def count_ge2(t1, t2):
            def cb(j, acc):
                a1, a2 = acc
                for u in range(4):
                    kv = _key(candv[pl.ds((j * 4 + u) * 16, 16)])
                    a1 = a1 + (kv >= t1).astype(jnp.int32)
                    a2 = a2 + (kv >= t2).astype(jnp.int32)
                return a1, a2
            p1, p2 = lax.fori_loop(
                0, nv4, cb,
                (jnp.zeros((16,), jnp.int32), jnp.zeros((16,), jnp.int32)))
            return (lax.reduce_max(plsc.cumsum(p1), axes=(0,)),
                    lax.reduce_max(plsc.cumsum(p2), axes=(0,)))

        def _cavg(a, b):
            return (a & b) + ((a ^ b) >> 1)

        lo0 = _scalar(_key(tl_vec)) - 1
        hi0 = _scalar(_key(jnp.broadcast_to(gmax, (16,)))) + 1

        def bs_cond(state):
            return state[0] < state[1]

        def bs_body(state):
            lo, hi, c_lo = state
            mid = _cavg(lo + 1, hi)
            m1 = _cavg(lo + 1, mid - 1)
            m1 = jnp.maximum(m1, lo + 1)
            c1, c2 = count_ge2(m1, mid)
            ge2 = c2 >= _K
            ge1 = c1 >= _K
            lo = jnp.where(ge2, mid, jnp.where(ge1, m1, lo))
            hi = jnp.where(ge2, hi, jnp.where(ge1, mid - 1, m1 - 1))
            c_lo = jnp.where(ge2, c2, jnp.where(ge1, c1, c_lo))
            return lo, hi, c_lo
        t, _unused, total = lax.while_loop(bs_cond, bs_body, (lo0, hi0, cnt))

        def count_ge(tq):
            def cb(j, acc):
                for u in range(4):
                    kv = _key(candv[pl.ds((j * 4 + u) * 16, 16)])
                    acc = acc + (kv >= tq).astype(jnp.int32)
                return acc
            part = lax.fori_loop(0, nv4, cb, jnp.zeros((16,), jnp.int32))
            return lax.reduce_max(plsc.cumsum(part), axes=(0,))

        fast = (total == _K) & (t != 0) & (t != -1)

        @pl.when(fast)
        def _():
            tf = _unkey_splat(t)

            def emit_fast(i, carry):
                for u in range(16):
                    sl = pl.ds((i * 16 + u) * 16, 16)
                    x = rowbuf[sl]
                    rowbuf[sl] = jnp.where(x >= tf, x, neg)
                return carry
            lax.fori_loop(0, _NV // 16, emit_fast, 0)

        @pl.when(jnp.logical_not(fast))
        def _():
            m_eq = _K - count_ge(t + 1)

            def emit(i, eq_seen):
                for u in range(4):
                    sl = pl.ds((i * 4 + u) * 16, 16)
                    x = rowbuf[sl]
                    kv = _key(x)
                    eq = kv == t
                    inc = plsc.cumsum(eq.astype(jnp.int32))
                    keep = (kv > t) | (eq & ((eq_seen + inc) <= m_eq))
                    rowbuf[sl] = jnp.where(keep, x, neg)
                    eq_seen = eq_seen + plsc.all_reduce_population_count(eq)[0]
                return eq_seen
            lax.fori_loop(0, _NV // 4, emit, jnp.int32(0))

        out_flight[r] = pltpu.async_copy(
            rowbuf, out_hbm.at[base_row + r], out_sems[r % 2])

    out_flight.pop(_ROWS_PER_TEC - 2).wait()
    out_flight.pop(_ROWS_PER_TEC - 1).wait()


def kernel(scores, k):
    mesh = plsc.VectorSubcoreMesh(core_axis_name="c", subcore_axis_name="s",
                                  num_cores=2, num_subcores=16)
    out = pl.kernel(
        _sc_body,
        out_type=jax.ShapeDtypeStruct((_R, _C), jnp.float32),
        mesh=mesh,
        compiler_params=pltpu.CompilerParams(needs_layout_passes=False),
        scratch_types=[
            pltpu.VMEM((_C,), jnp.float32),
            pltpu.VMEM((_C,), jnp.float32),
            pltpu.VMEM((_C + 64,), jnp.float32),
            pltpu.SemaphoreType.DMA,
            pltpu.SemaphoreType.DMA,
            pltpu.SemaphoreType.DMA,
            pltpu.SemaphoreType.DMA,
        ],
    )(scores)
    return out + (k * 0)
